# Initial kernel scaffold; baseline (speedup 1.0000x reference)
import jax
import jax.numpy as jnp
from jax.experimental import pallas as pl


def kernel(x, W1, b1, W2, batch_idx):
    raise NotImplementedError("write your pallas kernel here")


# speedup vs baseline: 1.3741x; 1.3741x over previous
import functools

import jax
import jax.numpy as jnp
from jax import lax
from jax.experimental import pallas as pl
from jax.experimental.pallas import tpu as pltpu
from jax.experimental.pallas import tpu_sc as plsc
import numpy as np

N_NODES = 100000
NODE_DIM = 512
HIDDEN = 256
N_GRAPHS = 512
_MEAN = 0.0
_STDDEV = 1.0
_LOG2 = float(np.log(2.0))


_BLK = 2000


def _mlp_body(x_ref, w1_ref, b1_ref, w2_ref, o_ref):
    h = jnp.dot(x_ref[...], w1_ref[...], preferred_element_type=jnp.float32)
    h = h + b1_ref[...]
    sp = jnp.maximum(h, 0.0) + jnp.log1p(jnp.exp(-jnp.abs(h))) - _LOG2
    o_ref[...] = jnp.dot(sp, w2_ref[...], preferred_element_type=jnp.float32)


def _mlp(x, W1, b1, W2):
    grid = (N_NODES // _BLK,)
    return pl.pallas_call(
        _mlp_body,
        grid=grid,
        in_specs=[
            pl.BlockSpec((_BLK, NODE_DIM), lambda i: (i, 0)),
            pl.BlockSpec((NODE_DIM, HIDDEN), lambda i: (0, 0)),
            pl.BlockSpec((1, HIDDEN), lambda i: (0, 0)),
            pl.BlockSpec((HIDDEN, 1), lambda i: (0, 0)),
        ],
        out_specs=pl.BlockSpec((_BLK, 1), lambda i: (i, 0)),
        out_shape=jax.ShapeDtypeStruct((N_NODES, 1), jnp.float32),
    )(x, W1, b1, W2)


_NS = 16
_PAD = 100096
_CHUNK = _PAD // _NS
_NVEC = _CHUNK // 16


def _seg_body(vals_hbm, idx_hbm, out_hbm, val_v, idx_v, acc2, accv, shared, gath):
    c = lax.axis_index("c")
    s = lax.axis_index("s")
    on = c == 0

    @pl.when(on)
    def _work():
        base = s * _CHUNK
        pltpu.sync_copy(vals_hbm.at[pl.ds(base, _CHUNK)], val_v)
        pltpu.sync_copy(idx_hbm.at[pl.ds(base, _CHUNK)], idx_v)

        def _zero(i, _):
            acc2[pl.ds(i * 16, 16)] = jnp.zeros((16,), jnp.float32)
            return 0

        lax.fori_loop(0, (16 * N_GRAPHS) // 16, _zero, 0)

        lane_off = lax.iota(jnp.int32, 16) * N_GRAPHS

        def _scat(i, _):
            v = val_v[pl.ds(i * 16, 16)]
            ix = idx_v[pl.ds(i * 16, 16)]
            plsc.addupdate_scatter(acc2, [lane_off + ix], v)
            return 0

        lax.fori_loop(0, _NVEC, _scat, 0)

        def _comb(ci, _):
            def _inner(r, t):
                return t + acc2[pl.ds(r * N_GRAPHS + ci * 16, 16)]

            t = lax.fori_loop(0, 16, _inner, jnp.zeros((16,), jnp.float32))
            accv[pl.ds(ci * 16, 16)] = t
            return 0

        lax.fori_loop(0, N_GRAPHS // 16, _comb, 0)
        pltpu.sync_copy(accv, shared.at[s])

    plsc.subcore_barrier()

    @pl.when(jnp.logical_and(on, s == 0))
    def _final():
        pltpu.sync_copy(shared, gath)

        def _fin(ci, _):
            def _inner(r, t):
                return t + gath[r, pl.ds(ci * 16, 16)]

            t = lax.fori_loop(0, _NS, _inner, jnp.zeros((16,), jnp.float32))
            accv[pl.ds(ci * 16, 16)] = t * _STDDEV + _MEAN
            return 0

        lax.fori_loop(0, N_GRAPHS // 16, _fin, 0)
        pltpu.sync_copy(accv, out_hbm)


def _segment_sum(vals, idx):
    mesh = plsc.VectorSubcoreMesh(core_axis_name="c", subcore_axis_name="s")
    f = pl.kernel(
        _seg_body,
        out_type=jax.ShapeDtypeStruct((N_GRAPHS,), jnp.float32),
        mesh=mesh,
        scratch_types=[
            pltpu.VMEM((_CHUNK,), jnp.float32),
            pltpu.VMEM((_CHUNK,), jnp.int32),
            pltpu.VMEM((_NS * N_GRAPHS,), jnp.float32),
            pltpu.VMEM((N_GRAPHS,), jnp.float32),
            pltpu.VMEM_SHARED((_NS, N_GRAPHS), jnp.float32),
            pltpu.VMEM((_NS, N_GRAPHS), jnp.float32),
        ],
        compiler_params=pltpu.CompilerParams(needs_layout_passes=False),
    )
    return f(vals, idx)


@functools.partial(jax.jit)
def kernel(x, W1, b1, W2, batch_idx):
    o = _mlp(x, W1, b1.reshape(1, HIDDEN), W2)
    vals = jnp.pad(o.reshape(N_NODES), (0, _PAD - N_NODES))
    idx = jnp.pad(batch_idx.astype(jnp.int32), (0, _PAD - N_NODES))
    agg = _segment_sum(vals, idx)
    return agg.reshape(N_GRAPHS, 1)
